# Initial kernel scaffold; baseline (speedup 1.0000x reference)
import jax
import jax.numpy as jnp
from jax.experimental import pallas as pl


def kernel(query, reference_points, value, Wv, bv, Ws, bs_, Wa, ba, Wo, bo):
    raise NotImplementedError("write your pallas kernel here")


# speedup vs baseline: 26.4907x; 26.4907x over previous
import functools
import math

import jax
import jax.numpy as jnp
import numpy as np
from jax import lax
from jax.experimental import pallas as pl
from jax.experimental.pallas import tpu as pltpu
from jax.experimental.pallas import tpu_sc as plsc

_SPATIAL_SHAPES = [(80, 80), (40, 40), (20, 20), (10, 10)]
_N_HEADS = 8
_EMBED = 256
_HEAD_DIM = 32
_P = 16
_B = 16
_LQ = 300
_LV = sum(h * w for h, w in _SPATIAL_SHAPES)
_BQ = _B * _LQ
_ROWS = _BQ * _N_HEADS

_LV_OFF = np.cumsum([0] + [h * w for h, w in _SPATIAL_SHAPES])[:4]
_WP = np.repeat([w for (_, w) in _SPATIAL_SHAPES], 4).astype(np.float32)
_HP = np.repeat([h for (h, _) in _SPATIAL_SHAPES], 4).astype(np.float32)
_WPI = _WP.astype(np.int32)
_OFFP = np.repeat(_LV_OFF, 4).astype(np.int32)

_NW = 32
_BQ_PER_W = _BQ // _NW
_ROW0_STEP = _BQ_PER_W * _N_HEADS


def _mm_kernel(x_ref, w_ref, b_ref, o_ref):
    o_ref[...] = (
        jnp.dot(x_ref[...], w_ref[...], preferred_element_type=jnp.float32)
        + b_ref[...]
    )


def _matmul_bias(x, w, b, bm):
    m, k = x.shape
    n = w.shape[1]
    return pl.pallas_call(
        _mm_kernel,
        grid=(m // bm,),
        in_specs=[
            pl.BlockSpec((bm, k), lambda i: (i, 0)),
            pl.BlockSpec((k, n), lambda i: (0, 0)),
            pl.BlockSpec((1, n), lambda i: (0, 0)),
        ],
        out_specs=pl.BlockSpec((bm, n), lambda i: (i, 0)),
        out_shape=jax.ShapeDtypeStruct((m, n), jnp.float32),
    )(x, w, b.reshape(1, n))


def _splat(val):
    return jnp.full((16,), val)


def _sc_body(s_hbm, rp_hbm, v_hbm, out_hbm, s_buf, rp_buf, idx_buf, w_buf,
             g_buf, out_buf, gsem):
    nc = 2
    wid = lax.axis_index("s") * nc + lax.axis_index("c")
    b = wid // 2
    vbase = b * (_LV * _N_HEADS)

    iota = lax.iota(jnp.int32, 16)
    level = lax.shift_right_logical(iota, 2)
    wpi = lax.shift_right_logical(jnp.full((16,), 80, jnp.int32), level)
    offp = jnp.where(
        level == 0, 0,
        jnp.where(level == 1, 6400, jnp.where(level == 2, 8000, 8400)))
    wp = wpi.astype(jnp.float32)
    hp = wp
    wm1 = wpi - 1
    hm1 = wm1

    def chunk(i, carry):
        bq = wid * _BQ_PER_W + i
        pltpu.sync_copy(s_hbm.at[bq], s_buf)
        pltpu.sync_copy(rp_hbm.at[bq // 4], rp_buf)
        u = (bq % 4) * 4
        rx = plsc.load_gather(rp_buf, [_splat(u)])
        ry = plsc.load_gather(rp_buf, [_splat(u + 1)])
        rw = plsc.load_gather(rp_buf, [_splat(u + 2)])
        rh = plsc.load_gather(rp_buf, [_splat(u + 3)])
        sclx = rw * 0.125
        scly = rh * 0.125

        for h in range(_N_HEADS):
            logits = s_buf[pl.ds(_EMBED + h * 16, 16)]
            mx = jnp.max(logits)
            e = jnp.exp(logits - mx)
            p = e / jnp.full((16,), jnp.sum(e))

            sx = plsc.load_gather(s_buf, [iota * 2 + (h * 32)])
            sy = plsc.load_gather(s_buf, [iota * 2 + (h * 32 + 1)])
            x = (rx + sx * sclx) * wp - 0.5
            y = (ry + sy * scly) * hp - 0.5
            xi = x.astype(jnp.int32)
            xf = xi.astype(jnp.float32)
            x0i = jnp.where(xf > x, xi - 1, xi)
            fx = x - jnp.where(xf > x, xf - 1.0, xf)
            yi = y.astype(jnp.int32)
            yf = yi.astype(jnp.float32)
            y0i = jnp.where(yf > y, yi - 1, yi)
            fy = y - jnp.where(yf > y, yf - 1.0, yf)
            gx = 1.0 - fx
            gy = 1.0 - fy
            x1i = x0i + 1
            y1i = y0i + 1
            vx0 = (x0i >= 0) & (x0i <= wm1)
            vx1 = (x1i >= 0) & (x1i <= wm1)
            vy0 = (y0i >= 0) & (y0i <= hm1)
            vy1 = (y1i >= 0) & (y1i <= hm1)
            cx0 = jnp.clip(x0i, 0, wm1)
            cx1 = jnp.clip(x1i, 0, wm1)
            cy0 = jnp.clip(y0i, 0, hm1)
            cy1 = jnp.clip(y1i, 0, hm1)
            r0 = offp + cy0 * wpi
            r1 = offp + cy1 * wpi
            base = vbase + h
            corners = (
                (r0 + cx0, gx * gy * (vx0 & vy0).astype(jnp.float32)),
                (r0 + cx1, fx * gy * (vx1 & vy0).astype(jnp.float32)),
                (r1 + cx0, gx * fy * (vx0 & vy1).astype(jnp.float32)),
                (r1 + cx1, fx * fy * (vx1 & vy1).astype(jnp.float32)),
            )
            for c, (ridx, wgt) in enumerate(corners):
                flat = h * 64 + c * 16
                idx_buf[flat >> 7, pl.ds(flat & 127, 16)] = ridx * 8 + base
                w_buf[pl.ds(flat, 16)] = wgt * p

        copies = [
            pltpu.async_copy(
                v_hbm.at[idx_buf.at[j]], g_buf.at[pl.ds(j * 128, 128)], gsem
            )
            for j in range(4)
        ]
        for cp in copies:
            cp.wait()

        def acc_row(r, carry):
            acc0 = jnp.zeros((16,), jnp.float32)
            acc1 = jnp.zeros((16,), jnp.float32)
            for j in range(64):
                n_ = r * 64 + j
                wj = plsc.load_gather(w_buf, [_splat(n_)])
                acc0 = acc0 + wj * g_buf[n_, pl.ds(0, 16)]
                acc1 = acc1 + wj * g_buf[n_, pl.ds(16, 16)]
            out_buf[r, pl.ds(0, 16)] = acc0
            out_buf[r, pl.ds(16, 16)] = acc1
            return carry

        lax.fori_loop(0, _N_HEADS, acc_row, 0)

        row0 = wid * _ROW0_STEP + i * _N_HEADS
        pltpu.sync_copy(out_buf, out_hbm.at[pl.ds(row0, _N_HEADS)])
        return carry

    lax.fori_loop(0, _BQ_PER_W, chunk, 0)


def _sc_sample(s_all, rp16, vrows):
    mesh = plsc.VectorSubcoreMesh(core_axis_name="c", subcore_axis_name="s")
    f = pl.kernel(
        _sc_body,
        out_type=jax.ShapeDtypeStruct((_ROWS, _HEAD_DIM), jnp.float32),
        mesh=mesh,
        compiler_params=pltpu.CompilerParams(
            needs_layout_passes=False, use_tc_tiling_on_sc=False),
        scratch_types=[
            pltpu.VMEM((384,), jnp.float32),
            pltpu.VMEM((16,), jnp.float32),
            pltpu.VMEM((4, 128), jnp.int32),
            pltpu.VMEM((512,), jnp.float32),
            pltpu.VMEM((512, _HEAD_DIM), jnp.float32),
            pltpu.VMEM((_N_HEADS, _HEAD_DIM), jnp.float32),
            pltpu.SemaphoreType.DMA,
        ],
    )
    return f(s_all, rp16, vrows)


def kernel(query, reference_points, value, Wv, bv, Ws, bs_, Wa, ba, Wo, bo):
    q2 = query.reshape(_BQ, _EMBED)
    v2 = value.reshape(_B * _LV, _EMBED)

    vmat = v2 @ Wv + bv
    wsa = jnp.concatenate([Ws, Wa], axis=1)
    bsa = jnp.concatenate([bs_, ba])
    s_all = q2 @ wsa + bsa

    rp16 = reference_points.reshape(_BQ // 4, 16)
    vrows = vmat.reshape(_B * _LV * _N_HEADS, _HEAD_DIM)
    sampled = _sc_sample(s_all, rp16, vrows)

    out = sampled.reshape(_BQ, _EMBED) @ Wo + bo
    return out.reshape(_B, _LQ, _EMBED)
